# baseline (speedup 1.0000x reference)
import functools
import jax
import jax.numpy as jnp
from jax import lax
from jax.experimental import pallas as pl
from jax.experimental.pallas import tpu as pltpu
from jax.experimental.pallas import tpu_sc as plsc

N = 10000
E = 320000
H = 128
NC = 2
NS = 16
NW = NC * NS
EPW = E // NW
CH = 80
NCH = EPW // CH
NP_PAD = 10240
NPT = NP_PAD // NS

_f32 = jnp.float32


def _elu(v):
    return jnp.where(v > 0, v, jnp.exp(jnp.minimum(v, 0.0)) - 1.0)


def _ln(v, g, b):
    mu = jnp.mean(v, axis=-1, keepdims=True)
    r = v - mu
    var = jnp.mean(r * r, axis=-1, keepdims=True)
    return r * jax.lax.rsqrt(var + 1e-5) * g + b


def _dot(a, b):
    return jnp.dot(a, b, preferred_element_type=_f32)


def _dot16(a, b):
    return jnp.dot(a.astype(jnp.bfloat16), b.astype(jnp.bfloat16),
                   preferred_element_type=_f32)


_MESH = plsc.VectorSubcoreMesh(core_axis_name="c", subcore_axis_name="s",
                               num_cores=NC, num_subcores=NS)


_RING = 3
_WRING = 2
_UNROLL = 6


def _gather_body(gs_hbm, gd_hbm, src_hbm, dst_hbm, ge_hbm,
                 srcv, dstv, bufs, bufd, bufo, *sems):
    sl = sems[:_RING]
    dl = sems[_RING:2 * _RING]
    wl = sems[2 * _RING:]
    c = lax.axis_index("c")
    s = lax.axis_index("s")
    wid = s * NC + c
    ebase = wid * EPW
    pltpu.sync_copy(src_hbm.at[wid], srcv)
    pltpu.sync_copy(dst_hbm.at[wid], dstv)

    def issue(b, ch):
        pltpu.make_async_copy(gs_hbm.at[srcv.at[ch]], bufs.at[b], sl[b]).start()
        pltpu.make_async_copy(gd_hbm.at[dstv.at[ch]], bufd.at[b], dl[b]).start()

    def drain(b, ch):
        pltpu.make_async_copy(gs_hbm.at[srcv.at[ch]], bufs.at[b], sl[b]).wait()
        pltpu.make_async_copy(gd_hbm.at[dstv.at[ch]], bufd.at[b], dl[b]).wait()

    def wstart(o, ch):
        off = ebase + ch * CH
        pltpu.make_async_copy(bufo.at[o], ge_hbm.at[pl.ds(off, CH)],
                              wl[o]).start()

    def wwait(o, ch):
        off = ebase + ch * CH
        pltpu.make_async_copy(bufo.at[o], ge_hbm.at[pl.ds(off, CH)],
                              wl[o]).wait()

    def addchunk(b, o):
        def addrow(r, carry):
            for k in range(H // 16):
                sel = pl.ds(16 * k, 16)
                bufo[o, r, sel] = bufs[b, r, sel] + bufd[b, r, sel]
            return carry

        lax.fori_loop(0, CH, addrow, 0)

    def chunk_step(b, o, ch):
        drain(b, ch)
        if isinstance(ch, int):
            if ch >= _WRING:
                wwait(o, ch - _WRING)
        else:
            @pl.when(ch >= _WRING)
            def _():
                wwait(o, ch - _WRING)
        addchunk(b, o)
        wstart(o, ch)
        if isinstance(ch, int):
            if ch + _RING < NCH:
                issue(b, ch + _RING)
        else:
            @pl.when(ch + _RING < NCH)
            def _():
                issue(b, ch + _RING)

    for b in range(_RING):
        issue(b, b)

    def step(jj, carry):
        for u in range(_UNROLL):
            ch = _UNROLL * jj + u
            chunk_step(u % _RING, u % _WRING, ch)
        return carry

    nmain = (NCH // _UNROLL) * _UNROLL
    lax.fori_loop(0, NCH // _UNROLL, step, 0)
    for t in range(nmain, NCH):
        chunk_step(t % _RING, t % _WRING, t)
    wwait((NCH - 2) % _WRING, NCH - 2)
    wwait((NCH - 1) % _WRING, NCH - 1)


def _gather(gs, gd, src3, dst3):
    return pl.kernel(
        _gather_body,
        out_type=jax.ShapeDtypeStruct((E, H), _f32),
        mesh=_MESH,
        scratch_types=(
            pltpu.VMEM((NCH, CH), jnp.int32),
            pltpu.VMEM((NCH, CH), jnp.int32),
            pltpu.VMEM((_RING, CH, H), _f32),
            pltpu.VMEM((_RING, CH, H), _f32),
            pltpu.VMEM((_WRING, CH, H), _f32),
        ) + (pltpu.SemaphoreType.DMA,) * (2 * _RING + _WRING),
    )(gs, gd, src3, dst3)


def _scatter_body(ea_hbm, dst_hbm, zrow_hbm, s_hbm, dstv, valA, valB,
                  semA, semB, acc):
    c = lax.axis_index("c")
    s = lax.axis_index("s")
    wid = s * NC + c
    ebase = wid * EPW
    nb = s * NPT
    pltpu.sync_copy(zrow_hbm, valA)
    for k in range(NPT // CH):
        pltpu.sync_copy(valA, acc.at[pl.ds(nb + CH * k, CH)])
    pltpu.sync_copy(dst_hbm.at[wid], dstv)
    plsc.subcore_barrier()

    def load(buf, sem, ch):
        pltpu.make_async_copy(
            ea_hbm.at[pl.ds(ebase + ch * CH, CH)], buf, sem).start()

    def drain(buf, sem, ch):
        pltpu.make_async_copy(
            ea_hbm.at[pl.ds(ebase + ch * CH, CH)], buf, sem).wait()

    load(valA, semA, 0)

    def step(jj, carry):
        chA = 2 * jj
        chB = 2 * jj + 1
        load(valB, semB, chB)
        drain(valA, semA, chA)
        pltpu.sync_copy(valA, acc.at[dstv.at[chA]], add=True)
        nxt = chA + 2

        @pl.when(nxt < NCH)
        def _():
            load(valA, semA, nxt)

        drain(valB, semB, chB)
        pltpu.sync_copy(valB, acc.at[dstv.at[chB]], add=True)
        return carry

    lax.fori_loop(0, NCH // 2, step, 0)
    drain(valA, semA, NCH - 1)
    pltpu.sync_copy(valA, acc.at[dstv.at[NCH - 1]], add=True)
    plsc.subcore_barrier()
    for k in range(NPT // CH):
        pltpu.sync_copy(acc.at[pl.ds(nb + CH * k, CH)], valA)
        pltpu.sync_copy(valA, s_hbm.at[c, pl.ds(nb + CH * k, CH)])


def _scatter(ea, dst3, zrow):
    return pl.kernel(
        _scatter_body,
        out_type=jax.ShapeDtypeStruct((NC, NP_PAD, H), _f32),
        mesh=_MESH,
        scratch_types=(
            pltpu.VMEM((NCH, CH), jnp.int32),
            pltpu.VMEM((CH, H), _f32),
            pltpu.VMEM((CH, H), _f32),
            pltpu.SemaphoreType.DMA,
            pltpu.SemaphoreType.DMA,
            pltpu.VMEM_SHARED((NP_PAD, H), _f32),
        ),
    )(ea, dst3, zrow)


def _counts_body(dst_hbm, zrow_hbm, ones_hbm, c_hbm, dstv, val, cacc):
    c = lax.axis_index("c")
    s = lax.axis_index("s")
    wid = s * NC + c
    nb = s * NPT
    pltpu.sync_copy(zrow_hbm, val)
    for k in range(NPT // CH):
        pltpu.sync_copy(val, cacc.at[pl.ds(nb + CH * k, CH)])
    pltpu.sync_copy(dst_hbm.at[wid], dstv)
    pltpu.sync_copy(ones_hbm, val)
    plsc.subcore_barrier()

    def step(j, carry):
        pltpu.sync_copy(val, cacc.at[dstv.at[j]], add=True)
        return carry

    lax.fori_loop(0, NCH, step, 0)
    plsc.subcore_barrier()
    for k in range(NPT // CH):
        pltpu.sync_copy(cacc.at[pl.ds(nb + CH * k, CH)], val)
        pltpu.sync_copy(val, c_hbm.at[c, pl.ds(nb + CH * k, CH)])


def _counts(dst3, zrow, ones):
    return pl.kernel(
        _counts_body,
        out_type=jax.ShapeDtypeStruct((NC, NP_PAD, H), _f32),
        mesh=_MESH,
        scratch_types=(
            pltpu.VMEM((NCH, CH), jnp.int32),
            pltpu.VMEM((CH, H), _f32),
            pltpu.VMEM_SHARED((NP_PAD, H), _f32),
        ),
    )(dst3, zrow, ones)


BN = 1000
BE = 4000


def _init_body(x_ref, wn_ref, bn_ref, ga_ref, gb_ref, x_out, gs_out, gd_out):
    x1 = _elu(_dot(x_ref[...], wn_ref[...]) + bn_ref[...])
    x_out[...] = x1
    gs_out[...] = _dot(x1, ga_ref[...])
    gd_out[...] = _dot(x1, gb_ref[...])


def _edge0_body(ea_ref, ge_ref, we_ref, be_ref, w0c_ref, eb0_ref,
                w1_ref, eb1_ref, g_ref, b_ref, out_ref):
    ea = _elu(_dot16(ea_ref[...], we_ref[...]) + be_ref[...])
    h = _elu(_dot16(ea, w0c_ref[...]) + ge_ref[...] + eb0_ref[...])
    et = _dot16(h, w1_ref[...]) + eb1_ref[...]
    out_ref[...] = _ln(ea + et, g_ref[...], b_ref[...])


def _edge1_body(ea_ref, ge_ref, w0c_ref, eb0_ref,
                w1_ref, eb1_ref, g_ref, b_ref, out_ref):
    ea = ea_ref[...]
    h = _elu(_dot16(ea, w0c_ref[...]) + ge_ref[...] + eb0_ref[...])
    et = _dot16(h, w1_ref[...]) + eb1_ref[...]
    out_ref[...] = _ln(ea + et, g_ref[...], b_ref[...])


def _node_core(x, s2, c2, w0a, w0b, nb0, w1, nb1, g, b):
    srow = s2[0] + s2[1]
    cnt = c2[0, :, 0:1] + c2[1, :, 0:1]
    agg = srow / jnp.maximum(cnt, 1.0)
    h = _elu(_dot(x, w0a) + _dot(agg, w0b) + nb0)
    xt = _dot(h, w1) + nb1
    return _ln(x + xt, g, b)


def _node_g_body(x_ref, s_ref, c_ref, w0a_ref, w0b_ref, nb0_ref, w1_ref,
                 nb1_ref, g_ref, b_ref, ga_ref, gb_ref,
                 x_out, gs_out, gd_out):
    xn = _node_core(x_ref[...], s_ref[...], c_ref[...], w0a_ref[...],
                    w0b_ref[...], nb0_ref[...], w1_ref[...], nb1_ref[...],
                    g_ref[...], b_ref[...])
    x_out[...] = xn
    gs_out[...] = _dot(xn, ga_ref[...])
    gd_out[...] = _dot(xn, gb_ref[...])


def _node_body(x_ref, s_ref, c_ref, w0a_ref, w0b_ref, nb0_ref, w1_ref,
               nb1_ref, g_ref, b_ref, x_out):
    x_out[...] = _node_core(x_ref[...], s_ref[...], c_ref[...], w0a_ref[...],
                            w0b_ref[...], nb0_ref[...], w1_ref[...],
                            nb1_ref[...], g_ref[...], b_ref[...])


_BS_W = pl.BlockSpec((H, H), lambda i: (0, 0))
_BS_B = pl.BlockSpec((1, H), lambda i: (0, 0))


def _init_call(x, wn, bn, ga, gb):
    bs_x = pl.BlockSpec((BN, H), lambda i: (i, 0))
    sds = jax.ShapeDtypeStruct((N, H), _f32)
    return pl.pallas_call(
        _init_body, grid=(N // BN,),
        in_specs=[bs_x, _BS_W, _BS_B, _BS_W, _BS_W],
        out_specs=[bs_x, bs_x, bs_x],
        out_shape=[sds, sds, sds])(x, wn, bn, ga, gb)


def _edge0_call(ea, ge, we, be, w0c, eb0, w1, eb1, g, b):
    bs_e = pl.BlockSpec((BE, H), lambda i: (i, 0))
    return pl.pallas_call(
        _edge0_body, grid=(E // BE,),
        in_specs=[bs_e, bs_e, _BS_W, _BS_B, _BS_W, _BS_B, _BS_W,
                  _BS_B, _BS_B, _BS_B],
        out_specs=bs_e,
        out_shape=jax.ShapeDtypeStruct((E, H), _f32))(
            ea, ge, we, be, w0c, eb0, w1, eb1, g, b)


def _edge1_call(ea, ge, w0c, eb0, w1, eb1, g, b):
    bs_e = pl.BlockSpec((BE, H), lambda i: (i, 0))
    return pl.pallas_call(
        _edge1_body, grid=(E // BE,),
        in_specs=[bs_e, bs_e, _BS_W, _BS_B, _BS_W, _BS_B, _BS_B,
                  _BS_B],
        out_specs=bs_e,
        out_shape=jax.ShapeDtypeStruct((E, H), _f32))(
            ea, ge, w0c, eb0, w1, eb1, g, b)


def _node_call(x, s2, c2, w0a, w0b, nb0, w1, nb1, g, b, ga=None, gb=None):
    bs_x = pl.BlockSpec((BN, H), lambda i: (i, 0))
    bs_s = pl.BlockSpec((NC, BN, H), lambda i: (0, i, 0))
    bs_c = pl.BlockSpec((NC, BN, H), lambda i: (0, i, 0))
    sds = jax.ShapeDtypeStruct((N, H), _f32)
    if ga is None:
        return pl.pallas_call(
            _node_body, grid=(N // BN,),
            in_specs=[bs_x, bs_s, bs_c, _BS_W, _BS_W, _BS_B, _BS_W, _BS_B,
                      _BS_B, _BS_B],
            out_specs=bs_x,
            out_shape=sds)(x, s2, c2, w0a, w0b, nb0, w1, nb1, g, b)
    return pl.pallas_call(
        _node_g_body, grid=(N // BN,),
        in_specs=[bs_x, bs_s, bs_c, _BS_W, _BS_W, _BS_B, _BS_W, _BS_B,
                  _BS_B, _BS_B, _BS_W, _BS_W],
        out_specs=[bs_x, bs_x, bs_x],
        out_shape=[sds, sds, sds])(x, s2, c2, w0a, w0b, nb0, w1, nb1, g, b,
                                   ga, gb)


def kernel(x, edge_index, edge_attr, pos, Wn, bn, We, be, eW0, eb0, eW1,
           eb1, eg, ebt, nW0, nb0, nW1, nb1, ng, nbt):
    del pos
    src3 = edge_index[0].astype(jnp.int32).reshape(NW, NCH, CH)
    dst3 = edge_index[1].astype(jnp.int32).reshape(NW, NCH, CH)
    zrow = jnp.zeros((CH, H), _f32)
    ones = jnp.ones((CH, H), _f32)

    r2 = lambda v: v.reshape(1, H)
    x1, gs, gd = _init_call(x, Wn, r2(bn), eW0[0, :H], eW0[0, H:2 * H])
    c2 = _counts(dst3, zrow, ones)

    ea = edge_attr
    for i in range(2):
        ge = _gather(gs, gd, src3, dst3)
        if i == 0:
            ea = _edge0_call(ea, ge, We, r2(be), eW0[0, 2 * H:],
                             r2(eb0[0]), eW1[0], r2(eb1[0]), r2(eg[0]),
                             r2(ebt[0]))
        else:
            ea = _edge1_call(ea, ge, eW0[1, 2 * H:], r2(eb0[1]),
                             eW1[1], r2(eb1[1]), r2(eg[1]), r2(ebt[1]))
        s2 = _scatter(ea, dst3, zrow)
        if i == 0:
            x1, gs, gd = _node_call(x1, s2, c2, nW0[0, :H], nW0[0, H:],
                                    r2(nb0[0]), nW1[0], r2(nb1[0]),
                                    r2(ng[0]), r2(nbt[0]),
                                    ga=eW0[1, :H], gb=eW0[1, H:2 * H])
        else:
            x1 = _node_call(x1, s2, c2, nW0[1, :H], nW0[1, H:], r2(nb0[1]),
                            nW1[1], r2(nb1[1]), r2(ng[1]), r2(nbt[1]))
    return x1


# speedup vs baseline: 1.0044x; 1.0044x over previous
import functools
import jax
import jax.numpy as jnp
from jax import lax
from jax.experimental import pallas as pl
from jax.experimental.pallas import tpu as pltpu
from jax.experimental.pallas import tpu_sc as plsc

N = 10000
E = 320000
H = 128
NC = 2
NS = 16
NW = NC * NS
EPW = E // NW
CH = 80
NCH = EPW // CH
NP_PAD = 10240
NPT = NP_PAD // NS

_f32 = jnp.float32


def _elu(v):
    return jnp.where(v > 0, v, jnp.exp(jnp.minimum(v, 0.0)) - 1.0)


def _ln(v, g, b):
    mu = jnp.mean(v, axis=-1, keepdims=True)
    r = v - mu
    var = jnp.mean(r * r, axis=-1, keepdims=True)
    return r * jax.lax.rsqrt(var + 1e-5) * g + b


def _dot(a, b):
    return jnp.dot(a, b, preferred_element_type=_f32)


_MESH = plsc.VectorSubcoreMesh(core_axis_name="c", subcore_axis_name="s",
                               num_cores=NC, num_subcores=NS)


_RING = 3
_WRING = 2
_UNROLL = 6


def _gather_body(gs_hbm, gd_hbm, src_hbm, dst_hbm, ge_hbm,
                 srcv, dstv, bufs, bufd, bufo, *sems):
    sl = sems[:_RING]
    dl = sems[_RING:2 * _RING]
    wl = sems[2 * _RING:]
    c = lax.axis_index("c")
    s = lax.axis_index("s")
    wid = s * NC + c
    ebase = wid * EPW
    pltpu.sync_copy(src_hbm.at[wid], srcv)
    pltpu.sync_copy(dst_hbm.at[wid], dstv)

    def issue(b, ch):
        pltpu.make_async_copy(gs_hbm.at[srcv.at[ch]], bufs.at[b], sl[b]).start()
        pltpu.make_async_copy(gd_hbm.at[dstv.at[ch]], bufd.at[b], dl[b]).start()

    def drain(b, ch):
        pltpu.make_async_copy(gs_hbm.at[srcv.at[ch]], bufs.at[b], sl[b]).wait()
        pltpu.make_async_copy(gd_hbm.at[dstv.at[ch]], bufd.at[b], dl[b]).wait()

    def wstart(o, ch):
        off = ebase + ch * CH
        pltpu.make_async_copy(bufo.at[o], ge_hbm.at[pl.ds(off, CH)],
                              wl[o]).start()

    def wwait(o, ch):
        off = ebase + ch * CH
        pltpu.make_async_copy(bufo.at[o], ge_hbm.at[pl.ds(off, CH)],
                              wl[o]).wait()

    def addchunk(b, o):
        def addrow(r, carry):
            for k in range(H // 16):
                sel = pl.ds(16 * k, 16)
                bufo[o, r, sel] = bufs[b, r, sel] + bufd[b, r, sel]
            return carry

        lax.fori_loop(0, CH, addrow, 0)

    def chunk_step(b, o, ch):
        drain(b, ch)
        if isinstance(ch, int):
            if ch >= _WRING:
                wwait(o, ch - _WRING)
        else:
            @pl.when(ch >= _WRING)
            def _():
                wwait(o, ch - _WRING)
        addchunk(b, o)
        wstart(o, ch)
        if isinstance(ch, int):
            if ch + _RING < NCH:
                issue(b, ch + _RING)
        else:
            @pl.when(ch + _RING < NCH)
            def _():
                issue(b, ch + _RING)

    for b in range(_RING):
        issue(b, b)

    def step(jj, carry):
        for u in range(_UNROLL):
            ch = _UNROLL * jj + u
            chunk_step(u % _RING, u % _WRING, ch)
        return carry

    nmain = (NCH // _UNROLL) * _UNROLL
    lax.fori_loop(0, NCH // _UNROLL, step, 0)
    for t in range(nmain, NCH):
        chunk_step(t % _RING, t % _WRING, t)
    wwait((NCH - 2) % _WRING, NCH - 2)
    wwait((NCH - 1) % _WRING, NCH - 1)


def _gather(gs, gd, src3, dst3):
    return pl.kernel(
        _gather_body,
        out_type=jax.ShapeDtypeStruct((E, H), _f32),
        mesh=_MESH,
        scratch_types=(
            pltpu.VMEM((NCH, CH), jnp.int32),
            pltpu.VMEM((NCH, CH), jnp.int32),
            pltpu.VMEM((_RING, CH, H), _f32),
            pltpu.VMEM((_RING, CH, H), _f32),
            pltpu.VMEM((_WRING, CH, H), _f32),
        ) + (pltpu.SemaphoreType.DMA,) * (2 * _RING + _WRING),
    )(gs, gd, src3, dst3)


def _scatter_body(ea_hbm, dst_hbm, zrow_hbm, s_hbm, dstv, valA, valB,
                  semA, semB, acc):
    c = lax.axis_index("c")
    s = lax.axis_index("s")
    wid = s * NC + c
    ebase = wid * EPW
    nb = s * NPT
    pltpu.sync_copy(zrow_hbm, valA)
    for k in range(NPT // CH):
        pltpu.sync_copy(valA, acc.at[pl.ds(nb + CH * k, CH)])
    pltpu.sync_copy(dst_hbm.at[wid], dstv)
    plsc.subcore_barrier()

    def load(buf, sem, ch):
        pltpu.make_async_copy(
            ea_hbm.at[pl.ds(ebase + ch * CH, CH)], buf, sem).start()

    def drain(buf, sem, ch):
        pltpu.make_async_copy(
            ea_hbm.at[pl.ds(ebase + ch * CH, CH)], buf, sem).wait()

    load(valA, semA, 0)

    def step(jj, carry):
        chA = 2 * jj
        chB = 2 * jj + 1
        load(valB, semB, chB)
        drain(valA, semA, chA)
        pltpu.sync_copy(valA, acc.at[dstv.at[chA]], add=True)
        nxt = chA + 2

        @pl.when(nxt < NCH)
        def _():
            load(valA, semA, nxt)

        drain(valB, semB, chB)
        pltpu.sync_copy(valB, acc.at[dstv.at[chB]], add=True)
        return carry

    lax.fori_loop(0, NCH // 2, step, 0)
    drain(valA, semA, NCH - 1)
    pltpu.sync_copy(valA, acc.at[dstv.at[NCH - 1]], add=True)
    plsc.subcore_barrier()
    for k in range(NPT // CH):
        pltpu.sync_copy(acc.at[pl.ds(nb + CH * k, CH)], valA)
        pltpu.sync_copy(valA, s_hbm.at[c, pl.ds(nb + CH * k, CH)])


def _scatter(ea, dst3, zrow):
    return pl.kernel(
        _scatter_body,
        out_type=jax.ShapeDtypeStruct((NC, NP_PAD, H), _f32),
        mesh=_MESH,
        scratch_types=(
            pltpu.VMEM((NCH, CH), jnp.int32),
            pltpu.VMEM((CH, H), _f32),
            pltpu.VMEM((CH, H), _f32),
            pltpu.SemaphoreType.DMA,
            pltpu.SemaphoreType.DMA,
            pltpu.VMEM_SHARED((NP_PAD, H), _f32),
        ),
    )(ea, dst3, zrow)


def _counts_body(dst_hbm, zrow_hbm, ones_hbm, c_hbm, dstv, val, cacc):
    c = lax.axis_index("c")
    s = lax.axis_index("s")
    wid = s * NC + c
    nb = s * NPT
    pltpu.sync_copy(zrow_hbm, val)
    for k in range(NPT // CH):
        pltpu.sync_copy(val, cacc.at[pl.ds(nb + CH * k, CH)])
    pltpu.sync_copy(dst_hbm.at[wid], dstv)
    pltpu.sync_copy(ones_hbm, val)
    plsc.subcore_barrier()

    def step(j, carry):
        pltpu.sync_copy(val, cacc.at[dstv.at[j]], add=True)
        return carry

    lax.fori_loop(0, NCH, step, 0)
    plsc.subcore_barrier()
    for k in range(NPT // CH):
        pltpu.sync_copy(cacc.at[pl.ds(nb + CH * k, CH)], val)
        pltpu.sync_copy(val, c_hbm.at[c, pl.ds(nb + CH * k, CH)])


def _counts(dst3, zrow, ones):
    return pl.kernel(
        _counts_body,
        out_type=jax.ShapeDtypeStruct((NC, NP_PAD, H), _f32),
        mesh=_MESH,
        scratch_types=(
            pltpu.VMEM((NCH, CH), jnp.int32),
            pltpu.VMEM((CH, H), _f32),
            pltpu.VMEM_SHARED((NP_PAD, H), _f32),
        ),
    )(dst3, zrow, ones)


BN = 1000
BE = 4000


def _init_body(x_ref, wn_ref, bn_ref, ga_ref, gb_ref, x_out, gs_out, gd_out):
    x1 = _elu(_dot(x_ref[...], wn_ref[...]) + bn_ref[...])
    x_out[...] = x1
    gs_out[...] = _dot(x1, ga_ref[...])
    gd_out[...] = _dot(x1, gb_ref[...])


def _edge0_body(ea_ref, ge_ref, we_ref, be_ref, w0c_ref, eb0_ref,
                w1_ref, eb1_ref, g_ref, b_ref, out_ref):
    ea = _elu(_dot(ea_ref[...], we_ref[...]) + be_ref[...])
    h = _elu(_dot(ea, w0c_ref[...]) + ge_ref[...] + eb0_ref[...])
    et = _dot(h, w1_ref[...]) + eb1_ref[...]
    out_ref[...] = _ln(ea + et, g_ref[...], b_ref[...])


def _edge1_body(ea_ref, ge_ref, w0c_ref, eb0_ref,
                w1_ref, eb1_ref, g_ref, b_ref, out_ref):
    ea = ea_ref[...]
    h = _elu(_dot(ea, w0c_ref[...]) + ge_ref[...] + eb0_ref[...])
    et = _dot(h, w1_ref[...]) + eb1_ref[...]
    out_ref[...] = _ln(ea + et, g_ref[...], b_ref[...])


def _node_core(x, s2, c2, w0a, w0b, nb0, w1, nb1, g, b):
    srow = s2[0] + s2[1]
    cnt = c2[0, :, 0:1] + c2[1, :, 0:1]
    agg = srow / jnp.maximum(cnt, 1.0)
    h = _elu(_dot(x, w0a) + _dot(agg, w0b) + nb0)
    xt = _dot(h, w1) + nb1
    return _ln(x + xt, g, b)


def _node_g_body(x_ref, s_ref, c_ref, w0a_ref, w0b_ref, nb0_ref, w1_ref,
                 nb1_ref, g_ref, b_ref, ga_ref, gb_ref,
                 x_out, gs_out, gd_out):
    xn = _node_core(x_ref[...], s_ref[...], c_ref[...], w0a_ref[...],
                    w0b_ref[...], nb0_ref[...], w1_ref[...], nb1_ref[...],
                    g_ref[...], b_ref[...])
    x_out[...] = xn
    gs_out[...] = _dot(xn, ga_ref[...])
    gd_out[...] = _dot(xn, gb_ref[...])


def _node_body(x_ref, s_ref, c_ref, w0a_ref, w0b_ref, nb0_ref, w1_ref,
               nb1_ref, g_ref, b_ref, x_out):
    x_out[...] = _node_core(x_ref[...], s_ref[...], c_ref[...], w0a_ref[...],
                            w0b_ref[...], nb0_ref[...], w1_ref[...],
                            nb1_ref[...], g_ref[...], b_ref[...])


_BS_W = pl.BlockSpec((H, H), lambda i: (0, 0))
_BS_B = pl.BlockSpec((1, H), lambda i: (0, 0))


def _init_call(x, wn, bn, ga, gb):
    bs_x = pl.BlockSpec((BN, H), lambda i: (i, 0))
    sds = jax.ShapeDtypeStruct((N, H), _f32)
    return pl.pallas_call(
        _init_body, grid=(N // BN,),
        in_specs=[bs_x, _BS_W, _BS_B, _BS_W, _BS_W],
        out_specs=[bs_x, bs_x, bs_x],
        out_shape=[sds, sds, sds])(x, wn, bn, ga, gb)


def _edge0_call(ea, ge, we, be, w0c, eb0, w1, eb1, g, b):
    bs_e = pl.BlockSpec((BE, H), lambda i: (i, 0))
    return pl.pallas_call(
        _edge0_body, grid=(E // BE,),
        in_specs=[bs_e, bs_e, _BS_W, _BS_B, _BS_W, _BS_B, _BS_W,
                  _BS_B, _BS_B, _BS_B],
        out_specs=bs_e,
        out_shape=jax.ShapeDtypeStruct((E, H), _f32))(
            ea, ge, we, be, w0c, eb0, w1, eb1, g, b)


def _edge1_call(ea, ge, w0c, eb0, w1, eb1, g, b):
    bs_e = pl.BlockSpec((BE, H), lambda i: (i, 0))
    return pl.pallas_call(
        _edge1_body, grid=(E // BE,),
        in_specs=[bs_e, bs_e, _BS_W, _BS_B, _BS_W, _BS_B, _BS_B,
                  _BS_B],
        out_specs=bs_e,
        out_shape=jax.ShapeDtypeStruct((E, H), _f32))(
            ea, ge, w0c, eb0, w1, eb1, g, b)


def _node_call(x, s2, c2, w0a, w0b, nb0, w1, nb1, g, b, ga=None, gb=None):
    bs_x = pl.BlockSpec((BN, H), lambda i: (i, 0))
    bs_s = pl.BlockSpec((NC, BN, H), lambda i: (0, i, 0))
    bs_c = pl.BlockSpec((NC, BN, H), lambda i: (0, i, 0))
    sds = jax.ShapeDtypeStruct((N, H), _f32)
    if ga is None:
        return pl.pallas_call(
            _node_body, grid=(N // BN,),
            in_specs=[bs_x, bs_s, bs_c, _BS_W, _BS_W, _BS_B, _BS_W, _BS_B,
                      _BS_B, _BS_B],
            out_specs=bs_x,
            out_shape=sds)(x, s2, c2, w0a, w0b, nb0, w1, nb1, g, b)
    return pl.pallas_call(
        _node_g_body, grid=(N // BN,),
        in_specs=[bs_x, bs_s, bs_c, _BS_W, _BS_W, _BS_B, _BS_W, _BS_B,
                  _BS_B, _BS_B, _BS_W, _BS_W],
        out_specs=[bs_x, bs_x, bs_x],
        out_shape=[sds, sds, sds])(x, s2, c2, w0a, w0b, nb0, w1, nb1, g, b,
                                   ga, gb)


def kernel(x, edge_index, edge_attr, pos, Wn, bn, We, be, eW0, eb0, eW1,
           eb1, eg, ebt, nW0, nb0, nW1, nb1, ng, nbt):
    del pos
    src3 = edge_index[0].astype(jnp.int32).reshape(NW, NCH, CH)
    dst3 = edge_index[1].astype(jnp.int32).reshape(NW, NCH, CH)
    zrow = jnp.zeros((CH, H), _f32)
    ones = jnp.ones((CH, H), _f32)

    r2 = lambda v: v.reshape(1, H)
    x1, gs, gd = _init_call(x, Wn, r2(bn), eW0[0, :H], eW0[0, H:2 * H])
    c2 = _counts(dst3, zrow, ones)

    ea = edge_attr
    for i in range(2):
        ge = _gather(gs, gd, src3, dst3)
        if i == 0:
            ea = _edge0_call(ea, ge, We, r2(be), eW0[0, 2 * H:],
                             r2(eb0[0]), eW1[0], r2(eb1[0]), r2(eg[0]),
                             r2(ebt[0]))
        else:
            ea = _edge1_call(ea, ge, eW0[1, 2 * H:], r2(eb0[1]),
                             eW1[1], r2(eb1[1]), r2(eg[1]), r2(ebt[1]))
        s2 = _scatter(ea, dst3, zrow)
        if i == 0:
            x1, gs, gd = _node_call(x1, s2, c2, nW0[0, :H], nW0[0, H:],
                                    r2(nb0[0]), nW1[0], r2(nb1[0]),
                                    r2(ng[0]), r2(nbt[0]),
                                    ga=eW0[1, :H], gb=eW0[1, H:2 * H])
        else:
            x1 = _node_call(x1, s2, c2, nW0[1, :H], nW0[1, H:], r2(nb0[1]),
                            nW1[1], r2(nb1[1]), r2(ng[1]), r2(nbt[1]))
    return x1
